# baseline (speedup 1.0000x reference)
import jax
import jax.numpy as jnp
from jax.experimental import pallas as pl
from jax.experimental.pallas import tpu as pltpu

_D = 2048
_MB1 = 256
_MN1 = _D // _MB1
_CH = 512
_CN = _D // _CH
_MB2 = 256
_MN2 = (_D // 2) // _MB2


def _pair_body(w3_ref, w2_ref, w1_ref, w0_ref, o_ref,
               rhsf_buf, rhsb_ref, lhs_buf, stage_ref,
               rhs_sem, lhs_sem, out_sem):
    s = pl.program_id(0)
    m = pl.program_id(1)

    def cp_rhs(c):
        slot = c % 2

        @pl.when(s == 0)
        def _():
            pltpu.make_async_copy(w2_ref.at[pl.ds(c * _CH, _CH), :],
                                  rhsf_buf.at[slot], rhs_sem.at[slot]).start()

        @pl.when(s == 1)
        def _():
            pltpu.make_async_copy(w0_ref.at[pl.ds(c * _CH, _CH), :],
                                  rhsf_buf.at[slot], rhs_sem.at[slot]).start()

    def cp_lhs(mm):
        slot = mm % 2

        @pl.when(s == 0)
        def _():
            pltpu.make_async_copy(w3_ref.at[pl.ds(mm * _MB1, _MB1), :],
                                  lhs_buf.at[slot], lhs_sem.at[slot]).start()

        @pl.when(s == 1)
        def _():
            pltpu.make_async_copy(w1_ref.at[pl.ds(mm * _MB1, _MB1), :],
                                  lhs_buf.at[slot], lhs_sem.at[slot]).start()

    @pl.when(m == 0)
    def _():
        cp_lhs(0)
        cp_rhs(0)
        cp_rhs(1)
        for c in range(_CN):
            pltpu.make_async_copy(rhsf_buf.at[c % 2], rhsf_buf.at[c % 2],
                                  rhs_sem.at[c % 2]).wait()
            rhsb_ref[pl.ds(c * _CH, _CH), :] = rhsf_buf[c % 2].astype(
                jnp.bfloat16)
            if c + 2 < _CN:
                cp_rhs(c + 2)

    @pl.when(m + 1 < _MN1)
    def _():
        cp_lhs(m + 1)

    slot = m % 2
    pltpu.make_async_copy(lhs_buf.at[slot], lhs_buf.at[slot],
                          lhs_sem.at[slot]).wait()

    @pl.when(m >= 2)
    def _():
        pltpu.make_async_copy(stage_ref.at[slot], stage_ref.at[slot],
                              out_sem.at[slot]).wait()

    res = jnp.dot(lhs_buf[slot].astype(jnp.bfloat16), rhsb_ref[...],
                  preferred_element_type=jnp.float32)
    stage_ref[slot] = res.astype(jnp.bfloat16)
    pltpu.make_async_copy(stage_ref.at[slot],
                          o_ref.at[s, pl.ds(m * _MB1, _MB1)],
                          out_sem.at[slot]).start()

    @pl.when(m == _MN1 - 1)
    def _():
        pltpu.make_async_copy(stage_ref.at[1 - slot], stage_ref.at[1 - slot],
                              out_sem.at[1 - slot]).wait()
        pltpu.make_async_copy(stage_ref.at[slot], stage_ref.at[slot],
                              out_sem.at[slot]).wait()


def _first_level(w0, w1, w2, w3):
    hbm = pl.BlockSpec(memory_space=pltpu.MemorySpace.HBM)
    return pl.pallas_call(
        _pair_body,
        out_shape=jax.ShapeDtypeStruct((2, _D, _D), jnp.bfloat16),
        grid=(2, _MN1),
        in_specs=[hbm] * 4,
        out_specs=hbm,
        scratch_shapes=[
            pltpu.VMEM((2, _CH, _D), jnp.float32),
            pltpu.VMEM((_D, _D), jnp.bfloat16),
            pltpu.VMEM((2, _MB1, _D), jnp.float32),
            pltpu.VMEM((2, _MB1, _D), jnp.bfloat16),
            pltpu.SemaphoreType.DMA((2,)),
            pltpu.SemaphoreType.DMA((2,)),
            pltpu.SemaphoreType.DMA((2,)),
        ],
        compiler_params=pltpu.CompilerParams(
            dimension_semantics=("parallel", "arbitrary"),
            vmem_limit_bytes=100 * 1024 * 1024),
    )(w3, w2, w1, w0)


def _final_body(ab_ref, o_ref, b_ref, lhs_buf, stage_ref,
                b_sem, lhs_sem, out_sem):
    i = pl.program_id(0)
    m = pl.program_id(1)

    def cp_lhs(mm):
        slot = mm % 2
        pltpu.make_async_copy(
            ab_ref.at[0, pl.ds(i * (_D // 2) + mm * _MB2, _MB2)],
            lhs_buf.at[slot], lhs_sem.at[slot]).start()

    @pl.when(m == 0)
    def _():
        cp_lhs(0)
        pltpu.make_async_copy(ab_ref.at[1], b_ref, b_sem.at[0]).start()
        pltpu.make_async_copy(b_ref, b_ref, b_sem.at[0]).wait()

    @pl.when(m + 1 < _MN2)
    def _():
        cp_lhs(m + 1)

    slot = m % 2
    pltpu.make_async_copy(lhs_buf.at[slot], lhs_buf.at[slot],
                          lhs_sem.at[slot]).wait()

    @pl.when(m >= 2)
    def _():
        pltpu.make_async_copy(stage_ref.at[slot], stage_ref.at[slot],
                              out_sem.at[slot]).wait()

    res = jnp.dot(lhs_buf[slot], b_ref[...],
                  preferred_element_type=jnp.float32)
    stage_ref[slot] = res
    pltpu.make_async_copy(
        stage_ref.at[slot],
        o_ref.at[pl.ds(i * (_D // 2) + m * _MB2, _MB2)],
        out_sem.at[slot]).start()

    @pl.when(m == _MN2 - 1)
    def _():
        pltpu.make_async_copy(stage_ref.at[1 - slot], stage_ref.at[1 - slot],
                              out_sem.at[1 - slot]).wait()
        pltpu.make_async_copy(stage_ref.at[slot], stage_ref.at[slot],
                              out_sem.at[slot]).wait()


def _final(ab):
    hbm = pl.BlockSpec(memory_space=pltpu.MemorySpace.HBM)
    return pl.pallas_call(
        _final_body,
        out_shape=jax.ShapeDtypeStruct((_D, _D), jnp.float32),
        grid=(2, _MN2),
        in_specs=[hbm],
        out_specs=hbm,
        scratch_shapes=[
            pltpu.VMEM((_D, _D), jnp.bfloat16),
            pltpu.VMEM((2, _MB2, _D), jnp.bfloat16),
            pltpu.VMEM((2, _MB2, _D), jnp.float32),
            pltpu.SemaphoreType.DMA((1,)),
            pltpu.SemaphoreType.DMA((2,)),
            pltpu.SemaphoreType.DMA((2,)),
        ],
        compiler_params=pltpu.CompilerParams(
            dimension_semantics=("parallel", "arbitrary"),
            vmem_limit_bytes=100 * 1024 * 1024),
    )(ab)


def kernel(w0, w1, w2, w3):
    ab = _first_level(w0, w1, w2, w3)
    return _final(ab)[None]


# speedup vs baseline: 1.0830x; 1.0830x over previous
import jax
import jax.numpy as jnp
from jax.experimental import pallas as pl
from jax.experimental.pallas import tpu as pltpu

_D = 2048
_KB = 512
_KN = _D // _KB
_BN2 = 1024


def _pair_body(w3_ref, w2_ref, w1_ref, w0_ref, o_ref,
               acc_ref, stage_ref, out_sem):
    s = pl.program_id(0)
    k = pl.program_id(1)

    @pl.when(k == 0)
    def _():
        acc_ref[...] = jnp.zeros_like(acc_ref)

    lhs = jnp.where(s == 0, w3_ref[...], w1_ref[...]).astype(jnp.bfloat16)
    rhs = jnp.where(s == 0, w2_ref[...], w0_ref[...]).astype(jnp.bfloat16)
    acc_ref[...] += jnp.dot(lhs, rhs, preferred_element_type=jnp.float32)

    @pl.when(k == _KN - 1)
    def _():
        stage_ref[...] = acc_ref[...].astype(jnp.bfloat16)
        cp = pltpu.make_async_copy(stage_ref, o_ref.at[s], out_sem)
        cp.start()
        cp.wait()


def _first_level(w0, w1, w2, w3):
    return pl.pallas_call(
        _pair_body,
        out_shape=jax.ShapeDtypeStruct((2, _D, _D), jnp.bfloat16),
        grid=(2, _KN),
        in_specs=[
            pl.BlockSpec((_D, _KB), lambda s, k: (0, jnp.where(s == 0, k, 0))),
            pl.BlockSpec((_KB, _D), lambda s, k: (jnp.where(s == 0, k, 0), 0)),
            pl.BlockSpec((_D, _KB), lambda s, k: (0, jnp.where(s == 1, k, 0))),
            pl.BlockSpec((_KB, _D), lambda s, k: (jnp.where(s == 1, k, 0), 0)),
        ],
        out_specs=pl.BlockSpec(memory_space=pltpu.MemorySpace.HBM),
        scratch_shapes=[
            pltpu.VMEM((_D, _D), jnp.float32),
            pltpu.VMEM((_D, _D), jnp.bfloat16),
            pltpu.SemaphoreType.DMA,
        ],
        compiler_params=pltpu.CompilerParams(
            dimension_semantics=("parallel", "arbitrary"),
            vmem_limit_bytes=100 * 1024 * 1024),
    )(w3, w2, w1, w0)


def _final_body(a_ref, b_ref, o_ref):
    o_ref[...] = jnp.dot(a_ref[0], b_ref[0],
                         preferred_element_type=jnp.float32)


def _final(ab):
    return pl.pallas_call(
        _final_body,
        out_shape=jax.ShapeDtypeStruct((_D, _D), jnp.float32),
        grid=(2, _D // _BN2),
        in_specs=[
            pl.BlockSpec((1, _D // 2, _D), lambda i, j: (0, i, 0)),
            pl.BlockSpec((1, _D, _BN2), lambda i, j: (1, 0, j)),
        ],
        out_specs=pl.BlockSpec((_D // 2, _BN2), lambda i, j: (i, j)),
        compiler_params=pltpu.CompilerParams(
            dimension_semantics=("parallel", "parallel"),
            vmem_limit_bytes=100 * 1024 * 1024),
    )(ab, ab)


def kernel(w0, w1, w2, w3):
    ab = _first_level(w0, w1, w2, w3)
    return _final(ab)[None]


# speedup vs baseline: 1.1189x; 1.0332x over previous
import jax
import jax.numpy as jnp
from jax.experimental import pallas as pl
from jax.experimental.pallas import tpu as pltpu

_D = 2048
_KB = 256
_KN = _D // _KB
_BN2 = 1024


def _pair_body(w3_ref, w2_ref, w1_ref, w0_ref, o_ref, acc_ref):
    s = pl.program_id(0)
    k = pl.program_id(1)

    @pl.when(k == 0)
    def _():
        acc_ref[...] = jnp.zeros_like(acc_ref)

    lhs = jnp.where(s == 0, w3_ref[...], w1_ref[...]).astype(jnp.bfloat16)
    rhs = jnp.where(s == 0, w2_ref[...], w0_ref[...]).astype(jnp.bfloat16)
    acc_ref[...] += jnp.dot(lhs, rhs, preferred_element_type=jnp.float32)

    @pl.when(k == _KN - 1)
    def _():
        o_ref[...] = acc_ref[...].astype(jnp.bfloat16)[None]


def _first_level(w0, w1, w2, w3):
    return pl.pallas_call(
        _pair_body,
        out_shape=jax.ShapeDtypeStruct((2, _D, _D), jnp.bfloat16),
        grid=(2, _KN),
        in_specs=[
            pl.BlockSpec((_D, _KB), lambda s, k: (0, jnp.where(s == 0, k, 0))),
            pl.BlockSpec((_KB, _D), lambda s, k: (jnp.where(s == 0, k, 0), 0)),
            pl.BlockSpec((_D, _KB), lambda s, k: (0, jnp.where(s == 1, k, 0))),
            pl.BlockSpec((_KB, _D), lambda s, k: (jnp.where(s == 1, k, 0), 0)),
        ],
        out_specs=pl.BlockSpec((1, _D, _D), lambda s, k: (s, 0, 0)),
        scratch_shapes=[pltpu.VMEM((_D, _D), jnp.float32)],
        compiler_params=pltpu.CompilerParams(
            dimension_semantics=("parallel", "arbitrary"),
            vmem_limit_bytes=100 * 1024 * 1024),
    )(w3, w2, w1, w0)


def _final_body(a_ref, b_ref, o_ref):
    o_ref[...] = jnp.dot(a_ref[0], b_ref[0],
                         preferred_element_type=jnp.float32)


def _final(ab):
    return pl.pallas_call(
        _final_body,
        out_shape=jax.ShapeDtypeStruct((_D, _D), jnp.float32),
        grid=(2, _D // _BN2),
        in_specs=[
            pl.BlockSpec((1, _D // 2, _D), lambda i, j: (0, i, 0)),
            pl.BlockSpec((1, _D, _BN2), lambda i, j: (1, 0, j)),
        ],
        out_specs=pl.BlockSpec((_D // 2, _BN2), lambda i, j: (i, j)),
        compiler_params=pltpu.CompilerParams(
            dimension_semantics=("parallel", "parallel"),
            vmem_limit_bytes=100 * 1024 * 1024),
    )(ab, ab)


def kernel(w0, w1, w2, w3):
    ab = _first_level(w0, w1, w2, w3)
    return _final(ab)[None]
